# Initial kernel scaffold; baseline (speedup 1.0000x reference)
import jax
import jax.numpy as jnp
from jax.experimental import pallas as pl


def kernel(x, ln1_g, c_attn_w, c_proj_w, ln2_g, w_g, c_fc, c_proj_e):
    raise NotImplementedError("write your pallas kernel here")


# speedup vs baseline: 1.6848x; 1.6848x over previous
import functools
import math

import jax
import jax.numpy as jnp
from jax import lax
from jax.experimental import pallas as pl
from jax.experimental.pallas import tpu as pltpu
from jax.experimental.pallas import tpu_sc as plsc

T = 2048
C = 1024
NH = 16
DH = 64
NE = 8
TOPK = 2
CAP = 640
TRASH = NE * CAP
DISP_ROWS = NE * CAP + 8
NPAIR = TOPK * T

BQ = 512
BR = 256

F32 = jnp.float32


def _ln(x, g):
    mu = jnp.mean(x, axis=1, keepdims=True)
    var = jnp.mean((x - mu) * (x - mu), axis=1, keepdims=True)
    return (x - mu) / jnp.sqrt(var + 1e-5) * g


def _ln1_qkv_body(x_ref, g_ref, w_ref, o_ref):
    h = _ln(x_ref[...], g_ref[...])
    o_ref[...] = lax.dot_general(h, w_ref[...], (((1,), (1,)), ((), ())),
                                 preferred_element_type=F32)


def _attn_body(q_ref, k_ref, v_ref, o_ref):
    j = pl.program_id(1)
    q = q_ref[0]
    k = k_ref[0]
    v = v_ref[0]
    s = lax.dot_general(q, k, (((1,), (1,)), ((), ())),
                        preferred_element_type=F32) * (1.0 / math.sqrt(DH))
    row = j * BQ + lax.broadcasted_iota(jnp.int32, (BQ, T), 0)
    col = lax.broadcasted_iota(jnp.int32, (BQ, T), 1)
    s = jnp.where(row >= col, s, -1e30)
    m = jnp.max(s, axis=1, keepdims=True)
    e = jnp.exp(s - m)
    p = e / jnp.sum(e, axis=1, keepdims=True)
    o_ref[0] = lax.dot_general(p, v, (((1,), (0,)), ((), ())),
                               preferred_element_type=F32)


def _proj_ln2_router_body(x_ref, y_ref, w_ref, g_ref, wg_ref,
                          x1_ref, h2_ref, lg_ref):
    x1 = x_ref[...] + lax.dot_general(y_ref[...], w_ref[...],
                                      (((1,), (1,)), ((), ())),
                                      preferred_element_type=F32)
    x1_ref[...] = x1
    h2 = _ln(x1, g_ref[...])
    h2_ref[...] = h2
    lg_ref[...] = lax.dot_general(h2, wg_ref[...], (((1,), (1,)), ((), ())),
                                  preferred_element_type=F32)


def _router_body(lg_ref, idx_ref, w_ref):
    l = lg_ref[...]
    col = lax.broadcasted_iota(jnp.int32, (T, NE), 1)
    v0 = jnp.max(l, axis=1, keepdims=True)
    i0 = jnp.min(jnp.where(l >= v0, col, NE), axis=1, keepdims=True)
    l2 = jnp.where(col == i0, -jnp.inf, l)
    v1 = jnp.max(l2, axis=1, keepdims=True)
    i1 = jnp.min(jnp.where(l2 >= v1, col, NE), axis=1, keepdims=True)
    e1 = jnp.exp(v1 - v0)
    p0 = 1.0 / (1.0 + e1)
    p1 = e1 / (1.0 + e1)
    oh0 = (col == i0).astype(F32)
    oh1 = (col == i1).astype(F32)
    ri = lax.broadcasted_iota(jnp.int32, (T, T), 0)
    ci = lax.broadcasted_iota(jnp.int32, (T, T), 1)
    tri = (ri > ci).astype(F32)
    c0 = lax.dot_general(tri, oh0, (((1,), (0,)), ((), ())),
                         preferred_element_type=F32)
    tot0 = jnp.sum(oh0, axis=0, keepdims=True)
    c1 = lax.dot_general(tri, oh1, (((1,), (0,)), ((), ())),
                         preferred_element_type=F32) + tot0
    rank0 = jnp.sum(oh0 * c0, axis=1, keepdims=True).astype(jnp.int32)
    rank1 = jnp.sum(oh1 * c1, axis=1, keepdims=True).astype(jnp.int32)
    valid0 = rank0 < CAP
    valid1 = rank1 < CAP
    dst0 = jnp.where(valid0, i0 * CAP + rank0, TRASH)
    dst1 = jnp.where(valid1, i1 * CAP + rank1, TRASH)
    src0 = i0 * CAP + jnp.minimum(rank0, CAP - 1)
    src1 = i1 * CAP + jnp.minimum(rank1, CAP - 1)
    w0 = jnp.where(valid0, p0, 0.0)
    w1 = jnp.where(valid1, p1, 0.0)
    zi = jnp.zeros_like(dst0)
    zf = jnp.zeros_like(w0)
    idx_ref[...] = jnp.concatenate(
        [dst0, dst1, src0, src1, zi, zi, zi, zi], axis=1)
    w_ref[...] = jnp.concatenate(
        [w0, w1, zf, zf, zf, zf, zf, zf], axis=1)


def _moe_mlp_body(xb_ref, fc_ref, pj_ref, o_ref):
    hb = pl.program_id(1)
    h = lax.dot_general(xb_ref[...], fc_ref[0], (((1,), (0,)), ((), ())),
                        preferred_element_type=F32)
    h = 0.5 * h * (1.0 + lax.erf(h * (1.0 / math.sqrt(2.0))))
    part = lax.dot_general(h, pj_ref[0], (((1,), (0,)), ((), ())),
                           preferred_element_type=F32)

    @pl.when(hb == 0)
    def _():
        o_ref[...] = part

    @pl.when(hb != 0)
    def _():
        o_ref[...] = o_ref[...] + part


def _combine_body(x1_ref, g0_ref, g1_ref, w_ref, o_ref):
    w = w_ref[...]
    o_ref[...] = (x1_ref[...] + w[:, 0:1] * g0_ref[...]
                  + w[:, 1:2] * g1_ref[...])


_NW = 32
_PW = NPAIR // _NW
_CH = 32
_NC = _PW // _CH


def _dispatch_body(h2_hbm, dst_hbm, out_hbm, idx_v, rows_v, sem):
    wid = lax.axis_index("s") * 2 + lax.axis_index("c")
    for c in range(_NC):
        base = wid * _PW + c * _CH
        nbase = lax.rem(base, T)
        pltpu.sync_copy(dst_hbm.at[pl.ds(base, _CH)], idx_v)
        pltpu.sync_copy(h2_hbm.at[pl.ds(nbase, _CH)], rows_v)
        pltpu.async_copy(rows_v, out_hbm.at[idx_v], sem).wait()


def _gather_body(tab_hbm, src_hbm, out_hbm, idx_v, rows_v, sem):
    wid = lax.axis_index("s") * 2 + lax.axis_index("c")
    for c in range(_NC):
        base = wid * _PW + c * _CH
        pltpu.sync_copy(src_hbm.at[pl.ds(base, _CH)], idx_v)
        pltpu.async_copy(tab_hbm.at[idx_v], rows_v, sem).wait()
        pltpu.sync_copy(rows_v, out_hbm.at[pl.ds(base, _CH)])


@functools.lru_cache(maxsize=None)
def _sc_kernels():
    mesh = plsc.VectorSubcoreMesh(core_axis_name="c", subcore_axis_name="s")
    scratch = [
        pltpu.VMEM((_CH,), jnp.int32),
        pltpu.VMEM((_CH, C), F32),
        pltpu.SemaphoreType.DMA,
    ]
    disp = functools.partial(
        pl.kernel, mesh=mesh,
        out_type=jax.ShapeDtypeStruct((DISP_ROWS, C), F32),
        scratch_types=scratch)(_dispatch_body)
    gath = functools.partial(
        pl.kernel, mesh=mesh,
        out_type=jax.ShapeDtypeStruct((NPAIR, C), F32),
        scratch_types=scratch)(_gather_body)
    return disp, gath


def _dispatch(h2, dst):
    return _sc_kernels()[0](h2, dst)


def _gather(tab, src):
    return _sc_kernels()[1](tab, src)


def kernel(x, ln1_g, c_attn_w, c_proj_w, ln2_g, w_g, c_fc, c_proj_e):
    x2 = x.reshape(T, C)
    g1 = ln1_g.reshape(1, C)
    g2 = ln2_g.reshape(1, C)

    qkv = pl.pallas_call(
        _ln1_qkv_body,
        grid=(T // BR,),
        in_specs=[pl.BlockSpec((BR, C), lambda i: (i, 0)),
                  pl.BlockSpec((1, C), lambda i: (0, 0)),
                  pl.BlockSpec((3 * C, C), lambda i: (0, 0))],
        out_specs=pl.BlockSpec((BR, 3 * C), lambda i: (i, 0)),
        out_shape=jax.ShapeDtypeStruct((T, 3 * C), F32),
    )(x2, g1, c_attn_w)

    q, k, v = jnp.split(qkv, 3, axis=1)
    q = q.reshape(T, NH, DH).transpose(1, 0, 2)
    k = k.reshape(T, NH, DH).transpose(1, 0, 2)
    v = v.reshape(T, NH, DH).transpose(1, 0, 2)

    y = pl.pallas_call(
        _attn_body,
        grid=(NH, T // BQ),
        in_specs=[pl.BlockSpec((1, BQ, DH), lambda h, j: (h, j, 0)),
                  pl.BlockSpec((1, T, DH), lambda h, j: (h, 0, 0)),
                  pl.BlockSpec((1, T, DH), lambda h, j: (h, 0, 0))],
        out_specs=pl.BlockSpec((1, BQ, DH), lambda h, j: (h, j, 0)),
        out_shape=jax.ShapeDtypeStruct((NH, T, DH), F32),
    )(q, k, v)

    y2 = y.transpose(1, 0, 2).reshape(T, C)

    x1, h2, logits = pl.pallas_call(
        _proj_ln2_router_body,
        grid=(T // BR,),
        in_specs=[pl.BlockSpec((BR, C), lambda i: (i, 0)),
                  pl.BlockSpec((BR, C), lambda i: (i, 0)),
                  pl.BlockSpec((C, C), lambda i: (0, 0)),
                  pl.BlockSpec((1, C), lambda i: (0, 0)),
                  pl.BlockSpec((NE, C), lambda i: (0, 0))],
        out_specs=[pl.BlockSpec((BR, C), lambda i: (i, 0)),
                   pl.BlockSpec((BR, C), lambda i: (i, 0)),
                   pl.BlockSpec((BR, NE), lambda i: (i, 0))],
        out_shape=[jax.ShapeDtypeStruct((T, C), F32),
                   jax.ShapeDtypeStruct((T, C), F32),
                   jax.ShapeDtypeStruct((T, NE), F32)],
    )(x2, y2, c_proj_w, g2, w_g)

    ridx, rw = pl.pallas_call(
        _router_body,
        out_shape=[jax.ShapeDtypeStruct((T, NE), jnp.int32),
                   jax.ShapeDtypeStruct((T, NE), F32)],
    )(logits)

    dst = jnp.concatenate([ridx[:, 0], ridx[:, 1]])
    src = jnp.concatenate([ridx[:, 2], ridx[:, 3]])

    exp_x = _dispatch(h2, dst)

    mlp = pl.pallas_call(
        _moe_mlp_body,
        grid=(NE, 4),
        in_specs=[pl.BlockSpec((CAP, C), lambda e, b: (e, 0)),
                  pl.BlockSpec((1, C, C), lambda e, b: (e, 0, b)),
                  pl.BlockSpec((1, C, C), lambda e, b: (e, b, 0))],
        out_specs=pl.BlockSpec((CAP, C), lambda e, b: (e, 0)),
        out_shape=jax.ShapeDtypeStruct((NE * CAP, C), F32),
    )(exp_x, c_fc, c_proj_e)

    g = _gather(mlp, src)

    out = pl.pallas_call(
        _combine_body,
        grid=(T // BR,),
        in_specs=[pl.BlockSpec((BR, C), lambda i: (i, 0)),
                  pl.BlockSpec((BR, C), lambda i: (i, 0)),
                  pl.BlockSpec((BR, C), lambda i: (i, 0)),
                  pl.BlockSpec((BR, NE), lambda i: (i, 0))],
        out_specs=pl.BlockSpec((BR, C), lambda i: (i, 0)),
        out_shape=jax.ShapeDtypeStruct((T, C), F32),
    )(x1, g[:T], g[T:], rw)

    return out.reshape(1, T, C)


# speedup vs baseline: 1.8736x; 1.1120x over previous
import functools
import math

import jax
import jax.numpy as jnp
from jax import lax
from jax.experimental import pallas as pl
from jax.experimental.pallas import tpu as pltpu
from jax.experimental.pallas import tpu_sc as plsc

T = 2048
C = 1024
NH = 16
DH = 64
NE = 8
TOPK = 2
CAP = 640
TRASH = NE * CAP
DISP_ROWS = NE * CAP + 8
NPAIR = TOPK * T

BQ = 512
BR = 256

F32 = jnp.float32


def _ln(x, g):
    mu = jnp.mean(x, axis=1, keepdims=True)
    var = jnp.mean((x - mu) * (x - mu), axis=1, keepdims=True)
    return (x - mu) / jnp.sqrt(var + 1e-5) * g


BF16 = jnp.bfloat16


def _ln1_qkv_body(x_ref, g_ref, w_ref, o_ref):
    h = _ln(x_ref[...], g_ref[...]).astype(BF16)
    o_ref[...] = lax.dot_general(w_ref[...].astype(BF16), h,
                                 (((1,), (1,)), ((), ())),
                                 preferred_element_type=F32)


def _attn_body(q_ref, k_ref, v_ref, o_ref):
    j = pl.program_id(1)
    q = q_ref[0].astype(BF16)
    k = k_ref[0].astype(BF16)
    v = v_ref[0].astype(BF16)
    s = lax.dot_general(q, k, (((0,), (0,)), ((), ())),
                        preferred_element_type=F32) * (1.0 / math.sqrt(DH))
    row = j * BQ + lax.broadcasted_iota(jnp.int32, (BQ, T), 0)
    col = lax.broadcasted_iota(jnp.int32, (BQ, T), 1)
    s = jnp.where(row >= col, s, -1e30)
    m = jnp.max(s, axis=1, keepdims=True)
    e = jnp.exp(s - m)
    p = (e / jnp.sum(e, axis=1, keepdims=True)).astype(BF16)
    o_ref[0] = lax.dot_general(p, v, (((1,), (1,)), ((), ())),
                               preferred_element_type=F32)


def _proj_ln2_router_body(x_ref, y_ref, w_ref, g_ref, wg_ref,
                          x1_ref, h2_ref, lg_ref):
    x1 = x_ref[...] + lax.dot_general(y_ref[...].astype(BF16),
                                      w_ref[...].astype(BF16),
                                      (((1,), (1,)), ((), ())),
                                      preferred_element_type=F32)
    x1_ref[...] = x1
    h2 = _ln(x1, g_ref[...])
    h2_ref[...] = h2
    lg_ref[...] = lax.dot_general(h2, wg_ref[...], (((1,), (1,)), ((), ())),
                                  preferred_element_type=F32)


def _router_body(lg_ref, idx_ref, w_ref):
    l = lg_ref[...]
    col = lax.broadcasted_iota(jnp.int32, (T, NE), 1)
    v0 = jnp.max(l, axis=1, keepdims=True)
    i0 = jnp.min(jnp.where(l >= v0, col, NE), axis=1, keepdims=True)
    l2 = jnp.where(col == i0, -jnp.inf, l)
    v1 = jnp.max(l2, axis=1, keepdims=True)
    i1 = jnp.min(jnp.where(l2 >= v1, col, NE), axis=1, keepdims=True)
    e1 = jnp.exp(v1 - v0)
    p0 = 1.0 / (1.0 + e1)
    p1 = e1 / (1.0 + e1)
    oh0 = (col == i0).astype(F32)
    oh1 = (col == i1).astype(F32)
    ri = lax.broadcasted_iota(jnp.int32, (T, T), 0)
    ci = lax.broadcasted_iota(jnp.int32, (T, T), 1)
    tri = (ri > ci).astype(F32)
    c0 = lax.dot_general(tri, oh0, (((1,), (0,)), ((), ())),
                         preferred_element_type=F32)
    tot0 = jnp.sum(oh0, axis=0, keepdims=True)
    c1 = lax.dot_general(tri, oh1, (((1,), (0,)), ((), ())),
                         preferred_element_type=F32) + tot0
    rank0 = jnp.sum(oh0 * c0, axis=1, keepdims=True).astype(jnp.int32)
    rank1 = jnp.sum(oh1 * c1, axis=1, keepdims=True).astype(jnp.int32)
    valid0 = rank0 < CAP
    valid1 = rank1 < CAP
    dst0 = jnp.where(valid0, i0 * CAP + rank0, TRASH)
    dst1 = jnp.where(valid1, i1 * CAP + rank1, TRASH)
    src0 = i0 * CAP + jnp.minimum(rank0, CAP - 1)
    src1 = i1 * CAP + jnp.minimum(rank1, CAP - 1)
    w0 = jnp.where(valid0, p0, 0.0)
    w1 = jnp.where(valid1, p1, 0.0)
    zi = jnp.zeros_like(dst0)
    zf = jnp.zeros_like(w0)
    idx_ref[...] = jnp.concatenate(
        [dst0, dst1, src0, src1, zi, zi, zi, zi], axis=1)
    w_ref[...] = jnp.concatenate(
        [w0, w1, zf, zf, zf, zf, zf, zf], axis=1)


def _moe_mlp_body(xb_ref, fc_ref, pj_ref, o_ref):
    hb = pl.program_id(1)
    h = lax.dot_general(xb_ref[...].astype(BF16), fc_ref[0].astype(BF16),
                        (((1,), (0,)), ((), ())),
                        preferred_element_type=F32)
    h = 0.5 * h * (1.0 + lax.erf(h * (1.0 / math.sqrt(2.0))))
    part = lax.dot_general(h.astype(BF16), pj_ref[0].astype(BF16),
                           (((1,), (0,)), ((), ())),
                           preferred_element_type=F32)

    @pl.when(hb == 0)
    def _():
        o_ref[...] = part

    @pl.when(hb != 0)
    def _():
        o_ref[...] = o_ref[...] + part


def _combine_body(x1_ref, g0_ref, g1_ref, w_ref, o_ref):
    w = w_ref[...]
    o_ref[...] = (x1_ref[...] + w[:, 0:1] * g0_ref[...]
                  + w[:, 1:2] * g1_ref[...])


_NW = 32
_PW = NPAIR // _NW
_CH = 32
_NC = _PW // _CH


def _dispatch_body(h2_hbm, dst_hbm, out_hbm, idx_v, rows_v, sem):
    wid = lax.axis_index("s") * 2 + lax.axis_index("c")
    for c in range(_NC):
        base = wid * _PW + c * _CH
        nbase = lax.rem(base, T)
        pltpu.sync_copy(dst_hbm.at[pl.ds(base, _CH)], idx_v)
        pltpu.sync_copy(h2_hbm.at[pl.ds(nbase, _CH)], rows_v)
        pltpu.async_copy(rows_v, out_hbm.at[idx_v], sem).wait()


def _gather_body(tab_hbm, src_hbm, out_hbm, idx_v, rows_v, sem):
    wid = lax.axis_index("s") * 2 + lax.axis_index("c")
    for c in range(_NC):
        base = wid * _PW + c * _CH
        pltpu.sync_copy(src_hbm.at[pl.ds(base, _CH)], idx_v)
        pltpu.async_copy(tab_hbm.at[idx_v], rows_v, sem).wait()
        pltpu.sync_copy(rows_v, out_hbm.at[pl.ds(base, _CH)])


@functools.lru_cache(maxsize=None)
def _sc_kernels():
    mesh = plsc.VectorSubcoreMesh(core_axis_name="c", subcore_axis_name="s")
    scratch = [
        pltpu.VMEM((_CH,), jnp.int32),
        pltpu.VMEM((_CH, C), F32),
        pltpu.SemaphoreType.DMA,
    ]
    disp = functools.partial(
        pl.kernel, mesh=mesh,
        out_type=jax.ShapeDtypeStruct((DISP_ROWS, C), F32),
        scratch_types=scratch)(_dispatch_body)
    gath = functools.partial(
        pl.kernel, mesh=mesh,
        out_type=jax.ShapeDtypeStruct((NPAIR, C), F32),
        scratch_types=scratch)(_gather_body)
    return disp, gath


def _dispatch(h2, dst):
    return _sc_kernels()[0](h2, dst)


def _gather(tab, src):
    return _sc_kernels()[1](tab, src)


def kernel(x, ln1_g, c_attn_w, c_proj_w, ln2_g, w_g, c_fc, c_proj_e):
    x2 = x.reshape(T, C)
    g1 = ln1_g.reshape(1, C)
    g2 = ln2_g.reshape(1, C)

    qkv_t = pl.pallas_call(
        _ln1_qkv_body,
        grid=(T // BR,),
        in_specs=[pl.BlockSpec((BR, C), lambda i: (i, 0)),
                  pl.BlockSpec((1, C), lambda i: (0, 0)),
                  pl.BlockSpec((3 * C, C), lambda i: (0, 0))],
        out_specs=pl.BlockSpec((3 * C, BR), lambda i: (0, i)),
        out_shape=jax.ShapeDtypeStruct((3 * C, T), F32),
    )(x2, g1, c_attn_w)

    q = qkv_t[:C].reshape(NH, DH, T)
    k = qkv_t[C:2 * C].reshape(NH, DH, T)
    v = qkv_t[2 * C:].reshape(NH, DH, T)

    y = pl.pallas_call(
        _attn_body,
        grid=(NH, T // BQ),
        in_specs=[pl.BlockSpec((1, DH, BQ), lambda h, j: (h, 0, j)),
                  pl.BlockSpec((1, DH, T), lambda h, j: (h, 0, 0)),
                  pl.BlockSpec((1, DH, T), lambda h, j: (h, 0, 0))],
        out_specs=pl.BlockSpec((1, BQ, DH), lambda h, j: (h, j, 0)),
        out_shape=jax.ShapeDtypeStruct((NH, T, DH), F32),
    )(q, k, v)

    y2 = y.transpose(1, 0, 2).reshape(T, C)

    x1, h2, logits = pl.pallas_call(
        _proj_ln2_router_body,
        grid=(T // BR,),
        in_specs=[pl.BlockSpec((BR, C), lambda i: (i, 0)),
                  pl.BlockSpec((BR, C), lambda i: (i, 0)),
                  pl.BlockSpec((C, C), lambda i: (0, 0)),
                  pl.BlockSpec((1, C), lambda i: (0, 0)),
                  pl.BlockSpec((NE, C), lambda i: (0, 0))],
        out_specs=[pl.BlockSpec((BR, C), lambda i: (i, 0)),
                   pl.BlockSpec((BR, C), lambda i: (i, 0)),
                   pl.BlockSpec((BR, NE), lambda i: (i, 0))],
        out_shape=[jax.ShapeDtypeStruct((T, C), F32),
                   jax.ShapeDtypeStruct((T, C), F32),
                   jax.ShapeDtypeStruct((T, NE), F32)],
    )(x2, y2, c_proj_w, g2, w_g)

    ridx, rw = pl.pallas_call(
        _router_body,
        out_shape=[jax.ShapeDtypeStruct((T, NE), jnp.int32),
                   jax.ShapeDtypeStruct((T, NE), F32)],
    )(logits)

    dst = jnp.concatenate([ridx[:, 0], ridx[:, 1]])
    src = jnp.concatenate([ridx[:, 2], ridx[:, 3]])

    exp_x = _dispatch(h2, dst)

    mlp = pl.pallas_call(
        _moe_mlp_body,
        grid=(NE, 4),
        in_specs=[pl.BlockSpec((CAP, C), lambda e, b: (e, 0)),
                  pl.BlockSpec((1, C, C), lambda e, b: (e, 0, b)),
                  pl.BlockSpec((1, C, C), lambda e, b: (e, b, 0))],
        out_specs=pl.BlockSpec((CAP, C), lambda e, b: (e, 0)),
        out_shape=jax.ShapeDtypeStruct((NE * CAP, C), F32),
    )(exp_x, c_fc, c_proj_e)

    g = _gather(mlp, src)

    out = pl.pallas_call(
        _combine_body,
        grid=(T // BR,),
        in_specs=[pl.BlockSpec((BR, C), lambda i: (i, 0)),
                  pl.BlockSpec((BR, C), lambda i: (i, 0)),
                  pl.BlockSpec((BR, C), lambda i: (i, 0)),
                  pl.BlockSpec((BR, NE), lambda i: (i, 0))],
        out_specs=pl.BlockSpec((BR, C), lambda i: (i, 0)),
        out_shape=jax.ShapeDtypeStruct((T, C), F32),
    )(x1, g[:T], g[T:], rw)

    return out.reshape(1, T, C)


# speedup vs baseline: 1.8860x; 1.0066x over previous
import functools
import math

import jax
import jax.numpy as jnp
from jax import lax
from jax.experimental import pallas as pl
from jax.experimental.pallas import tpu as pltpu
from jax.experimental.pallas import tpu_sc as plsc

T = 2048
C = 1024
NH = 16
DH = 64
NE = 8
TOPK = 2
CAP = 640
TRASH = NE * CAP
DISP_ROWS = NE * CAP + 8
NPAIR = TOPK * T

BQ = 512
BR = 256

F32 = jnp.float32


def _ln(x, g):
    mu = jnp.mean(x, axis=1, keepdims=True)
    var = jnp.mean((x - mu) * (x - mu), axis=1, keepdims=True)
    return (x - mu) / jnp.sqrt(var + 1e-5) * g


BF16 = jnp.bfloat16


def _ln1_qkv_body(x_ref, g_ref, w_ref, o_ref):
    h = _ln(x_ref[...], g_ref[...])
    o_ref[...] = lax.dot_general(w_ref[...], h,
                                 (((1,), (1,)), ((), ())),
                                 preferred_element_type=F32)


def _attn_body(q_ref, k_ref, v_ref, o_ref):
    j = pl.program_id(1)
    q = q_ref[0]
    k = k_ref[0]
    v = v_ref[0]
    s = lax.dot_general(q, k, (((0,), (0,)), ((), ())),
                        preferred_element_type=F32) * (1.0 / math.sqrt(DH))
    row = j * BQ + lax.broadcasted_iota(jnp.int32, (BQ, T), 0)
    col = lax.broadcasted_iota(jnp.int32, (BQ, T), 1)
    s = jnp.where(row >= col, s, -1e30)
    m = jnp.max(s, axis=1, keepdims=True)
    e = jnp.exp(s - m)
    p = e / jnp.sum(e, axis=1, keepdims=True)
    o_ref[0] = lax.dot_general(p, v, (((1,), (1,)), ((), ())),
                               preferred_element_type=F32)


def _proj_ln2_router_body(x_ref, y_ref, w_ref, g_ref, wg_ref,
                          x1_ref, h2_ref, lg_ref):
    x1 = x_ref[...] + lax.dot_general(y_ref[...], w_ref[...],
                                      (((1,), (1,)), ((), ())),
                                      preferred_element_type=F32)
    x1_ref[...] = x1
    h2 = _ln(x1, g_ref[...])
    h2_ref[...] = h2
    lg_ref[...] = lax.dot_general(h2, wg_ref[...], (((1,), (1,)), ((), ())),
                                  preferred_element_type=F32)


def _router_body(lg_ref, idx_ref, w_ref):
    l = lg_ref[...]
    col = lax.broadcasted_iota(jnp.int32, (T, NE), 1)
    v0 = jnp.max(l, axis=1, keepdims=True)
    i0 = jnp.min(jnp.where(l >= v0, col, NE), axis=1, keepdims=True)
    l2 = jnp.where(col == i0, -jnp.inf, l)
    v1 = jnp.max(l2, axis=1, keepdims=True)
    i1 = jnp.min(jnp.where(l2 >= v1, col, NE), axis=1, keepdims=True)
    e1 = jnp.exp(v1 - v0)
    p0 = 1.0 / (1.0 + e1)
    p1 = e1 / (1.0 + e1)
    oh0 = (col == i0).astype(F32)
    oh1 = (col == i1).astype(F32)
    ri = lax.broadcasted_iota(jnp.int32, (T, T), 0)
    ci = lax.broadcasted_iota(jnp.int32, (T, T), 1)
    tri = (ri > ci).astype(F32)
    c0 = lax.dot_general(tri, oh0, (((1,), (0,)), ((), ())),
                         preferred_element_type=F32)
    tot0 = jnp.sum(oh0, axis=0, keepdims=True)
    c1 = lax.dot_general(tri, oh1, (((1,), (0,)), ((), ())),
                         preferred_element_type=F32) + tot0
    rank0 = jnp.sum(oh0 * c0, axis=1, keepdims=True).astype(jnp.int32)
    rank1 = jnp.sum(oh1 * c1, axis=1, keepdims=True).astype(jnp.int32)
    valid0 = rank0 < CAP
    valid1 = rank1 < CAP
    dst0 = jnp.where(valid0, i0 * CAP + rank0, TRASH)
    dst1 = jnp.where(valid1, i1 * CAP + rank1, TRASH)
    src0 = i0 * CAP + jnp.minimum(rank0, CAP - 1)
    src1 = i1 * CAP + jnp.minimum(rank1, CAP - 1)
    w0 = jnp.where(valid0, p0, 0.0)
    w1 = jnp.where(valid1, p1, 0.0)
    zi = jnp.zeros_like(dst0)
    zf = jnp.zeros_like(w0)
    idx_ref[...] = jnp.concatenate(
        [dst0, dst1, src0, src1, zi, zi, zi, zi], axis=1)
    w_ref[...] = jnp.concatenate(
        [w0, w1, zf, zf, zf, zf, zf, zf], axis=1)


def _moe_mlp_body(xb_ref, fc_ref, pj_ref, o_ref):
    hb = pl.program_id(1)
    h = lax.dot_general(xb_ref[...].astype(BF16), fc_ref[0].astype(BF16),
                        (((1,), (0,)), ((), ())),
                        preferred_element_type=F32)
    h = 0.5 * h * (1.0 + lax.erf(h * (1.0 / math.sqrt(2.0))))
    part = lax.dot_general(h.astype(BF16), pj_ref[0].astype(BF16),
                           (((1,), (0,)), ((), ())),
                           preferred_element_type=F32)

    @pl.when(hb == 0)
    def _():
        o_ref[...] = part

    @pl.when(hb != 0)
    def _():
        o_ref[...] = o_ref[...] + part


def _combine_body(x1_ref, g0_ref, g1_ref, w_ref, o_ref):
    w = w_ref[...]
    o_ref[...] = (x1_ref[...] + w[:, 0:1] * g0_ref[...]
                  + w[:, 1:2] * g1_ref[...])


_NW = 32
_PW = NPAIR // _NW
_CH = 32
_NC = _PW // _CH


def _dispatch_body(h2_hbm, dst_hbm, out_hbm, idx_v, rows_v, sem):
    wid = lax.axis_index("s") * 2 + lax.axis_index("c")
    for c in range(_NC):
        base = wid * _PW + c * _CH
        nbase = lax.rem(base, T)
        pltpu.sync_copy(dst_hbm.at[pl.ds(base, _CH)], idx_v)
        pltpu.sync_copy(h2_hbm.at[pl.ds(nbase, _CH)], rows_v)
        pltpu.async_copy(rows_v, out_hbm.at[idx_v], sem).wait()


def _gather_body(tab_hbm, src_hbm, out_hbm, idx_v, rows_v, sem):
    wid = lax.axis_index("s") * 2 + lax.axis_index("c")
    for c in range(_NC):
        base = wid * _PW + c * _CH
        pltpu.sync_copy(src_hbm.at[pl.ds(base, _CH)], idx_v)
        pltpu.async_copy(tab_hbm.at[idx_v], rows_v, sem).wait()
        pltpu.sync_copy(rows_v, out_hbm.at[pl.ds(base, _CH)])


@functools.lru_cache(maxsize=None)
def _sc_kernels():
    mesh = plsc.VectorSubcoreMesh(core_axis_name="c", subcore_axis_name="s")
    scratch = [
        pltpu.VMEM((_CH,), jnp.int32),
        pltpu.VMEM((_CH, C), F32),
        pltpu.SemaphoreType.DMA,
    ]
    disp = functools.partial(
        pl.kernel, mesh=mesh,
        out_type=jax.ShapeDtypeStruct((DISP_ROWS, C), F32),
        scratch_types=scratch)(_dispatch_body)
    gath = functools.partial(
        pl.kernel, mesh=mesh,
        out_type=jax.ShapeDtypeStruct((NPAIR, C), F32),
        scratch_types=scratch)(_gather_body)
    return disp, gath


def _dispatch(h2, dst):
    return _sc_kernels()[0](h2, dst)


def _gather(tab, src):
    return _sc_kernels()[1](tab, src)


def kernel(x, ln1_g, c_attn_w, c_proj_w, ln2_g, w_g, c_fc, c_proj_e):
    x2 = x.reshape(T, C)
    g1 = ln1_g.reshape(1, C)
    g2 = ln2_g.reshape(1, C)

    qkv_t = pl.pallas_call(
        _ln1_qkv_body,
        grid=(T // BR,),
        in_specs=[pl.BlockSpec((BR, C), lambda i: (i, 0)),
                  pl.BlockSpec((1, C), lambda i: (0, 0)),
                  pl.BlockSpec((3 * C, C), lambda i: (0, 0))],
        out_specs=pl.BlockSpec((3 * C, BR), lambda i: (0, i)),
        out_shape=jax.ShapeDtypeStruct((3 * C, T), F32),
    )(x2, g1, c_attn_w)

    q = qkv_t[:C].reshape(NH, DH, T)
    k = qkv_t[C:2 * C].reshape(NH, DH, T)
    v = qkv_t[2 * C:].reshape(NH, DH, T)

    y = pl.pallas_call(
        _attn_body,
        grid=(NH, T // BQ),
        in_specs=[pl.BlockSpec((1, DH, BQ), lambda h, j: (h, 0, j)),
                  pl.BlockSpec((1, DH, T), lambda h, j: (h, 0, 0)),
                  pl.BlockSpec((1, DH, T), lambda h, j: (h, 0, 0))],
        out_specs=pl.BlockSpec((1, BQ, DH), lambda h, j: (h, j, 0)),
        out_shape=jax.ShapeDtypeStruct((NH, T, DH), F32),
    )(q, k, v)

    y2 = y.transpose(1, 0, 2).reshape(T, C)

    x1, h2, logits = pl.pallas_call(
        _proj_ln2_router_body,
        grid=(T // BR,),
        in_specs=[pl.BlockSpec((BR, C), lambda i: (i, 0)),
                  pl.BlockSpec((BR, C), lambda i: (i, 0)),
                  pl.BlockSpec((C, C), lambda i: (0, 0)),
                  pl.BlockSpec((1, C), lambda i: (0, 0)),
                  pl.BlockSpec((NE, C), lambda i: (0, 0))],
        out_specs=[pl.BlockSpec((BR, C), lambda i: (i, 0)),
                   pl.BlockSpec((BR, C), lambda i: (i, 0)),
                   pl.BlockSpec((BR, NE), lambda i: (i, 0))],
        out_shape=[jax.ShapeDtypeStruct((T, C), F32),
                   jax.ShapeDtypeStruct((T, C), F32),
                   jax.ShapeDtypeStruct((T, NE), F32)],
    )(x2, y2, c_proj_w, g2, w_g)

    ridx, rw = pl.pallas_call(
        _router_body,
        out_shape=[jax.ShapeDtypeStruct((T, NE), jnp.int32),
                   jax.ShapeDtypeStruct((T, NE), F32)],
    )(logits)

    dst = jnp.concatenate([ridx[:, 0], ridx[:, 1]])
    src = jnp.concatenate([ridx[:, 2], ridx[:, 3]])

    exp_x = _dispatch(h2, dst)

    mlp = pl.pallas_call(
        _moe_mlp_body,
        grid=(NE, 4),
        in_specs=[pl.BlockSpec((CAP, C), lambda e, b: (e, 0)),
                  pl.BlockSpec((1, C, C), lambda e, b: (e, 0, b)),
                  pl.BlockSpec((1, C, C), lambda e, b: (e, b, 0))],
        out_specs=pl.BlockSpec((CAP, C), lambda e, b: (e, 0)),
        out_shape=jax.ShapeDtypeStruct((NE * CAP, C), F32),
    )(exp_x, c_fc, c_proj_e)

    g = _gather(mlp, src)

    out = pl.pallas_call(
        _combine_body,
        grid=(T // BR,),
        in_specs=[pl.BlockSpec((BR, C), lambda i: (i, 0)),
                  pl.BlockSpec((BR, C), lambda i: (i, 0)),
                  pl.BlockSpec((BR, C), lambda i: (i, 0)),
                  pl.BlockSpec((BR, NE), lambda i: (i, 0))],
        out_specs=pl.BlockSpec((BR, C), lambda i: (i, 0)),
        out_shape=jax.ShapeDtypeStruct((T, C), F32),
    )(x1, g[:T], g[T:], rw)

    return out.reshape(1, T, C)
